# baseline (speedup 1.0000x reference)
import functools

import numpy as np
import jax
import jax.numpy as jnp
from jax import lax
from jax.experimental import pallas as pl
from jax.experimental.pallas import tpu as pltpu
from jax.experimental.pallas import tpu_sc as plsc

COUNTS_ = 2000
LANE_ = 128
ROUNDS_ = 3


@functools.lru_cache(maxsize=None)
def _choice_indices(anchors_num: int) -> np.ndarray:
    rng = np.random.default_rng(0)
    return rng.choice(np.arange(anchors_num), size=COUNTS_, replace=False)


@functools.lru_cache(maxsize=None)
def _make_stream_gather(batches: int, coords: int, anchors_num: int,
                        p_cell: int, e_pad: int):
    info = plsc.get_sparse_core_info()
    nc, ns, nl = info.num_cores, info.num_subcores, info.num_lanes
    nw = nc * ns
    full_tiles = anchors_num // LANE_
    total_tiles = batches * full_tiles
    per_w = -(-total_tiles // nw)
    tpr = -(-per_w // ROUNDS_)
    mesh = plsc.VectorSubcoreMesh(core_axis_name="c", subcore_axis_name="s")

    @functools.partial(
        pl.kernel,
        mesh=mesh,
        out_type=jax.ShapeDtypeStruct((e_pad,), jnp.float32),
        compiler_params=pltpu.CompilerParams(needs_layout_passes=False),
        scratch_types=[
            pltpu.VMEM((tpr * coords, LANE_), jnp.float32),
            pltpu.VMEM((p_cell // LANE_, LANE_), jnp.int32),
            pltpu.VMEM((p_cell // LANE_, LANE_), jnp.int32),
            pltpu.VMEM((p_cell,), jnp.float32),
            pltpu.SemaphoreType.DMA,
            pltpu.SemaphoreType.DMA,
        ],
    )
    def stream_kernel(tab3d, sidx_hbm, pos_hbm, out_hbm,
                      ring, sidx_v, pos_v, vals_v, sem_in, sem_out):
        wid = lax.axis_index("s") * nc + lax.axis_index("c")
        for r in range(ROUNDS_):
            def fire(i, _):
                g = wid * per_w + r * tpr + i

                @pl.when(g < total_tiles)
                def _():
                    b = g // full_tiles
                    t = g % full_tiles
                    pltpu.async_copy(
                        tab3d.at[b, pl.ds(0, coords),
                                 pl.ds(pl.multiple_of(t * LANE_, LANE_),
                                       LANE_)],
                        ring.at[pl.ds(pl.multiple_of(i * coords, coords),
                                      coords)],
                        sem_in)
                return ()

            lax.fori_loop(0, tpr, fire, (), unroll=False)
            row = wid * ROUNDS_ + r
            pltpu.sync_copy(sidx_hbm.at[row], sidx_v)
            pltpu.sync_copy(pos_hbm.at[row], pos_v)

            def drain(i, _):
                g = wid * per_w + r * tpr + i

                @pl.when(g < total_tiles)
                def _():
                    pltpu.make_async_copy(
                        tab3d.at[0, pl.ds(0, coords), pl.ds(0, LANE_)],
                        ring.at[pl.ds(0, coords)], sem_in).wait()
                return ()

            lax.fori_loop(0, tpr, drain, (), unroll=False)

            for v in range(p_cell // nl):
                lv = sidx_v[v // 8, pl.ds((v % 8) * nl, nl)]
                rown = jnp.right_shift(lv, 7)
                lane = jnp.bitwise_and(lv, LANE_ - 1)
                vals_v[pl.ds(v * nl, nl)] = plsc.load_gather(
                    ring, [rown, lane])

            cps = [
                pltpu.async_copy(
                    vals_v.at[pl.ds(ch * LANE_, LANE_)],
                    out_hbm.at[pos_v.at[ch]], sem_out)
                for ch in range(p_cell // LANE_)
            ]
            for cp in cps:
                cp.wait()

    return stream_kernel


def kernel(feature_maps, anchors):
    anc = anchors[0]
    batches, anchors_num, coords = anc.shape
    full_tiles = anchors_num // LANE_
    total_tiles = batches * full_tiles
    info = plsc.get_sparse_core_info()
    nw = info.num_cores * info.num_subcores
    per_w = -(-total_tiles // nw)
    tpr = -(-per_w // ROUNDS_)
    idx = np.asarray(_choice_indices(anchors_num)).astype(np.int64)

    b_arr = np.repeat(np.arange(batches, dtype=np.int64), COUNTS_ * coords)
    j_arr = np.tile(np.repeat(np.arange(COUNTS_, dtype=np.int64), coords),
                    batches)
    c_arr = np.tile(np.arange(coords, dtype=np.int64), batches * COUNTS_)
    a_arr = idx[j_arr]
    if np.any(a_arr >= full_tiles * LANE_):
        raise NotImplementedError(
            "sampled anchor in ragged tail tile; not reachable for the "
            "operation's deterministic sample")
    e = b_arr.shape[0]
    g_el = b_arr * full_tiles + a_arr // LANE_
    w_el = g_el // per_w
    local = g_el % per_w
    r_el = local // tpr
    slot = local % tpr
    packed = (slot * coords + c_arr) * LANE_ + a_arr % LANE_
    cell = w_el * ROUNDS_ + r_el
    counts = np.bincount(cell, minlength=nw * ROUNDS_)
    p_cell = max(LANE_, int(-(-counts.max() // LANE_) * LANE_))
    e_pad = -(-e // (nw * LANE_)) * (nw * LANE_)

    sidx = np.zeros((nw * ROUNDS_, p_cell), dtype=np.int32)
    pos = np.full((nw * ROUNDS_, p_cell), e_pad - 8, dtype=np.int32)
    order = np.argsort(cell, kind="stable")
    off = np.zeros(nw * ROUNDS_, dtype=np.int64)
    cello = cell[order]
    starts = np.searchsorted(cello, np.arange(nw * ROUNDS_))
    ends = np.searchsorted(cello, np.arange(nw * ROUNDS_) + 1)
    for ci in range(nw * ROUNDS_):
        ks = order[starts[ci]:ends[ci]]
        sidx[ci, :len(ks)] = packed[ks]
        pos[ci, :len(ks)] = ks
    del off

    t3d = jnp.transpose(anc, (0, 2, 1))
    out = _make_stream_gather(batches, coords, anchors_num, p_cell, e_pad)(
        t3d,
        jnp.asarray(sidx.reshape(nw * ROUNDS_, p_cell // LANE_, LANE_)),
        jnp.asarray(pos.reshape(nw * ROUNDS_, p_cell // LANE_, LANE_)))
    return out[:e].reshape(batches, COUNTS_, coords)


# speedup vs baseline: 1.0041x; 1.0041x over previous
import functools

import numpy as np
import jax
import jax.numpy as jnp
from jax import lax
from jax.experimental import pallas as pl
from jax.experimental.pallas import tpu as pltpu
from jax.experimental.pallas import tpu_sc as plsc

COUNTS_ = 2000
LANE_ = 128
ROUNDS_ = 3


@functools.lru_cache(maxsize=None)
def _choice_indices(anchors_num: int) -> np.ndarray:
    rng = np.random.default_rng(0)
    return rng.choice(np.arange(anchors_num), size=COUNTS_, replace=False)


@functools.lru_cache(maxsize=None)
def _make_stream_gather(batches: int, coords: int, anchors_num: int,
                        p_cell: int, e_pad: int):
    info = plsc.get_sparse_core_info()
    nc, ns, nl = info.num_cores, info.num_subcores, info.num_lanes
    nw = nc * ns
    full_tiles = anchors_num // LANE_
    total_tiles = batches * full_tiles
    per_w = -(-total_tiles // nw)
    tpr = -(-per_w // ROUNDS_)
    mesh = plsc.VectorSubcoreMesh(core_axis_name="c", subcore_axis_name="s")

    @functools.partial(
        pl.kernel,
        mesh=mesh,
        out_type=jax.ShapeDtypeStruct((e_pad,), jnp.float32),
        compiler_params=pltpu.CompilerParams(needs_layout_passes=False),
        scratch_types=[
            pltpu.VMEM((tpr * coords, LANE_), jnp.float32),
            pltpu.VMEM((p_cell // LANE_, LANE_), jnp.int32),
            pltpu.VMEM((p_cell // LANE_, LANE_), jnp.int32),
            pltpu.VMEM((p_cell,), jnp.float32),
            pltpu.SemaphoreType.DMA,
            pltpu.SemaphoreType.DMA,
        ],
    )
    def stream_kernel(tab3d, sidx_hbm, pos_hbm, out_hbm,
                      ring, sidx_v, pos_v, vals_v, sem_in, sem_out):
        wid = lax.axis_index("s") * nc + lax.axis_index("c")
        for r in range(ROUNDS_):
            def fire(i, _):
                g = wid * per_w + r * tpr + i

                @pl.when(g < total_tiles)
                def _():
                    b = sum((g >= (bb + 1) * full_tiles).astype(jnp.int32)
                            for bb in range(batches - 1))
                    t = g - b * full_tiles
                    pltpu.async_copy(
                        tab3d.at[b, pl.ds(0, coords),
                                 pl.ds(pl.multiple_of(t * LANE_, LANE_),
                                       LANE_)],
                        ring.at[pl.ds(pl.multiple_of(i * coords, coords),
                                      coords)],
                        sem_in)
                return ()

            lax.fori_loop(0, tpr, fire, (), unroll=False)
            row = wid * ROUNDS_ + r
            pltpu.sync_copy(sidx_hbm.at[row], sidx_v)
            pltpu.sync_copy(pos_hbm.at[row], pos_v)

            def drain(i, _):
                g = wid * per_w + r * tpr + i

                @pl.when(g < total_tiles)
                def _():
                    pltpu.make_async_copy(
                        tab3d.at[0, pl.ds(0, coords), pl.ds(0, LANE_)],
                        ring.at[pl.ds(0, coords)], sem_in).wait()
                return ()

            lax.fori_loop(0, tpr, drain, (), unroll=False)

            for v in range(p_cell // nl):
                lv = sidx_v[v // 8, pl.ds((v % 8) * nl, nl)]
                rown = jnp.right_shift(lv, 7)
                lane = jnp.bitwise_and(lv, LANE_ - 1)
                vals_v[pl.ds(v * nl, nl)] = plsc.load_gather(
                    ring, [rown, lane])

            cps = [
                pltpu.async_copy(
                    vals_v.at[pl.ds(ch * LANE_, LANE_)],
                    out_hbm.at[pos_v.at[ch]], sem_out)
                for ch in range(p_cell // LANE_)
            ]
            for cp in cps:
                cp.wait()

    return stream_kernel


def kernel(feature_maps, anchors):
    anc = anchors[0]
    batches, anchors_num, coords = anc.shape
    full_tiles = anchors_num // LANE_
    total_tiles = batches * full_tiles
    info = plsc.get_sparse_core_info()
    nw = info.num_cores * info.num_subcores
    per_w = -(-total_tiles // nw)
    tpr = -(-per_w // ROUNDS_)
    idx = np.asarray(_choice_indices(anchors_num)).astype(np.int64)

    b_arr = np.repeat(np.arange(batches, dtype=np.int64), COUNTS_ * coords)
    j_arr = np.tile(np.repeat(np.arange(COUNTS_, dtype=np.int64), coords),
                    batches)
    c_arr = np.tile(np.arange(coords, dtype=np.int64), batches * COUNTS_)
    a_arr = idx[j_arr]
    if np.any(a_arr >= full_tiles * LANE_):
        raise NotImplementedError(
            "sampled anchor in ragged tail tile; not reachable for the "
            "operation's deterministic sample")
    e = b_arr.shape[0]
    g_el = b_arr * full_tiles + a_arr // LANE_
    w_el = g_el // per_w
    local = g_el % per_w
    r_el = local // tpr
    slot = local % tpr
    packed = (slot * coords + c_arr) * LANE_ + a_arr % LANE_
    cell = w_el * ROUNDS_ + r_el
    counts = np.bincount(cell, minlength=nw * ROUNDS_)
    p_cell = max(LANE_, int(-(-counts.max() // LANE_) * LANE_))
    e_pad = -(-e // (nw * LANE_)) * (nw * LANE_)

    sidx = np.zeros((nw * ROUNDS_, p_cell), dtype=np.int32)
    pos = np.full((nw * ROUNDS_, p_cell), e_pad - 8, dtype=np.int32)
    order = np.argsort(cell, kind="stable")
    off = np.zeros(nw * ROUNDS_, dtype=np.int64)
    cello = cell[order]
    starts = np.searchsorted(cello, np.arange(nw * ROUNDS_))
    ends = np.searchsorted(cello, np.arange(nw * ROUNDS_) + 1)
    for ci in range(nw * ROUNDS_):
        ks = order[starts[ci]:ends[ci]]
        sidx[ci, :len(ks)] = packed[ks]
        pos[ci, :len(ks)] = ks
    del off

    t3d = jnp.transpose(anc, (0, 2, 1))
    out = _make_stream_gather(batches, coords, anchors_num, p_cell, e_pad)(
        t3d,
        jnp.asarray(sidx.reshape(nw * ROUNDS_, p_cell // LANE_, LANE_)),
        jnp.asarray(pos.reshape(nw * ROUNDS_, p_cell // LANE_, LANE_)))
    return out[:e].reshape(batches, COUNTS_, coords)


# speedup vs baseline: 40.7837x; 40.6169x over previous
import functools

import numpy as np
import jax
import jax.numpy as jnp
from jax import lax
from jax.experimental import pallas as pl
from jax.experimental.pallas import tpu as pltpu
from jax.experimental.pallas import tpu_sc as plsc

COUNTS_ = 2000
LANE_ = 128
CHUNK_ = 128
NBUF_ = 16


@functools.lru_cache(maxsize=None)
def _choice_indices(anchors_num: int) -> np.ndarray:
    rng = np.random.default_rng(0)
    return rng.choice(np.arange(anchors_num), size=COUNTS_, replace=False)


@functools.lru_cache(maxsize=None)
def _make_retile(batches: int, coords: int, anchors_num: int):
    info = plsc.get_sparse_core_info()
    nc, ns = info.num_cores, info.num_subcores
    nw = nc * ns
    full_tiles = anchors_num // LANE_
    pairs_per_b = full_tiles // 2
    npairs = batches * pairs_per_b
    per_w = -(-npairs // nw)
    rows = batches * full_tiles * coords
    assert rows % 8 == 0
    mesh = plsc.VectorSubcoreMesh(core_axis_name="c", subcore_axis_name="s")

    @functools.partial(
        pl.kernel,
        mesh=mesh,
        out_type=jax.ShapeDtypeStruct((rows, LANE_), jnp.float32),
        compiler_params=pltpu.CompilerParams(needs_layout_passes=False),
        scratch_types=(
            [pltpu.VMEM((2 * coords, LANE_), jnp.float32)] * NBUF_
            + [pltpu.SemaphoreType.DMA, pltpu.SemaphoreType.DMA]
        ),
    )
    def retile_kernel(tab3d, out_hbm, *rest):
        bufs = rest[:NBUF_]
        sem_in, sem_out = rest[NBUF_:]
        wid = lax.axis_index("s") * nc + lax.axis_index("c")
        g0 = wid * per_w

        def block(i0, _):
            for j in range(NBUF_):
                g = g0 + i0 + j

                @pl.when(g < npairs)
                def _():
                    b = g // pairs_per_b
                    t2 = g % pairs_per_b
                    a0 = pl.multiple_of(t2 * (2 * LANE_), 2 * LANE_)
                    pltpu.async_copy(
                        tab3d.at[b, pl.ds(0, coords), pl.ds(a0, LANE_)],
                        bufs[j].at[pl.ds(0, coords)], sem_in)
                    pltpu.async_copy(
                        tab3d.at[b, pl.ds(0, coords),
                                 pl.ds(a0 + LANE_, LANE_)],
                        bufs[j].at[pl.ds(coords, coords)], sem_in)

            for j in range(NBUF_):
                g = g0 + i0 + j

                @pl.when(g < npairs)
                def _():
                    pltpu.make_async_copy(
                        tab3d.at[0, pl.ds(0, coords), pl.ds(0, LANE_)],
                        bufs[j].at[pl.ds(0, coords)], sem_in).wait()
                    pltpu.make_async_copy(
                        tab3d.at[0, pl.ds(0, coords), pl.ds(0, LANE_)],
                        bufs[j].at[pl.ds(coords, coords)], sem_in).wait()
                    r0 = pl.multiple_of(g * (2 * coords), 2 * coords)
                    pltpu.async_copy(
                        bufs[j], out_hbm.at[pl.ds(r0, 2 * coords)], sem_out)

            for j in range(NBUF_):
                g = g0 + i0 + j

                @pl.when(g < npairs)
                def _():
                    pltpu.make_async_copy(
                        out_hbm.at[pl.ds(0, 2 * coords)], bufs[j],
                        sem_out).wait()
            return ()

        lax.fori_loop(0, -(-per_w // NBUF_),
                      lambda k, c: block(k * NBUF_, c), (), unroll=False)

    return retile_kernel


@functools.lru_cache(maxsize=None)
def _make_gather(num_elems: int, e_pad: int, fixups: tuple):
    info = plsc.get_sparse_core_info()
    nc, ns, nl = info.num_cores, info.num_subcores, info.num_lanes
    nw = nc * ns
    e_per_w = e_pad // nw
    chunks = e_per_w // CHUNK_
    mesh = plsc.VectorSubcoreMesh(core_axis_name="c", subcore_axis_name="s")

    @functools.partial(
        pl.kernel,
        mesh=mesh,
        out_type=jax.ShapeDtypeStruct((e_pad,), jnp.float32),
        compiler_params=pltpu.CompilerParams(needs_layout_passes=False),
        scratch_types=[
            pltpu.VMEM((chunks, CHUNK_), jnp.int32),
            pltpu.VMEM((e_per_w,), jnp.float32),
            pltpu.VMEM((nl,), jnp.float32),
            pltpu.SemaphoreType.DMA,
        ],
    )
    def gather_kernel(table_hbm, fidx_hbm, tab3d, out_hbm,
                      fidx_v, vals_v, stage_v, sem):
        wid = lax.axis_index("s") * nc + lax.axis_index("c")
        pltpu.sync_copy(fidx_hbm.at[pl.ds(wid * chunks, chunks)], fidx_v)
        copies = [
            pltpu.async_copy(
                table_hbm.at[fidx_v.at[c]],
                vals_v.at[pl.ds(c * CHUNK_, CHUNK_)], sem)
            for c in range(chunks)
        ]
        for cp in copies:
            cp.wait()
        for (k, b, c, a) in fixups:
            owner, vec, lane = k // e_per_w, (k % e_per_w) // nl, k % nl

            @pl.when(wid == owner)
            def _():
                pltpu.sync_copy(tab3d.at[b, c, pl.ds(a, 1)],
                                stage_v.at[pl.ds(0, 1)])
                sv = stage_v[pl.ds(0, nl)]
                dn = lax.GatherDimensionNumbers(
                    offset_dims=(), collapsed_slice_dims=(0,),
                    start_index_map=(0,))
                bc = lax.gather(
                    sv, jnp.zeros((nl, 1), jnp.int32), dn, (1,),
                    mode=lax.GatherScatterMode.PROMISE_IN_BOUNDS)
                old = vals_v[pl.ds(vec * nl, nl)]
                sel = lax.iota(jnp.int32, nl) == lane
                vals_v[pl.ds(vec * nl, nl)] = jnp.where(sel, bc, old)
        pltpu.sync_copy(vals_v, out_hbm.at[pl.ds(wid * e_per_w, e_per_w)])

    return gather_kernel


def kernel(feature_maps, anchors):
    anc = anchors[0]
    batches, anchors_num, coords = anc.shape
    full_tiles = anchors_num // LANE_
    idx = np.asarray(_choice_indices(anchors_num)).astype(np.int64)

    b_arr = np.repeat(np.arange(batches, dtype=np.int64), COUNTS_ * coords)
    j_arr = np.tile(np.repeat(np.arange(COUNTS_, dtype=np.int64), coords),
                    batches)
    c_arr = np.tile(np.arange(coords, dtype=np.int64), batches * COUNTS_)
    a_arr = idx[j_arr]
    main = a_arr < full_tiles * LANE_
    fidx = np.where(
        main,
        ((b_arr * full_tiles + a_arr // LANE_) * coords + c_arr) * LANE_
        + a_arr % LANE_,
        0,
    )
    fixups = tuple(
        (int(k), int(b_arr[k]), int(c_arr[k]), int(a_arr[k]))
        for k in np.nonzero(~main)[0]
    )
    e = fidx.shape[0]
    info = plsc.get_sparse_core_info()
    nw = info.num_cores * info.num_subcores
    e_pad = -(-e // (nw * CHUNK_)) * (nw * CHUNK_)
    fidx_pad = np.zeros((e_pad,), dtype=np.int32)
    fidx_pad[:e] = fidx

    t3d = jnp.transpose(anc, (0, 2, 1))
    dense = _make_retile(batches, coords, anchors_num)(t3d)
    out = _make_gather(dense.shape[0] * LANE_, e_pad, fixups)(
        dense.reshape(-1), jnp.asarray(fidx_pad.reshape(-1, CHUNK_)), t3d)
    return out[:e].reshape(batches, COUNTS_, coords)
